# baseline (speedup 1.0000x reference)
import functools

import jax
import jax.numpy as jnp
from jax import lax
from jax.experimental import pallas as pl
from jax.experimental.pallas import tpu as pltpu
from jax.experimental.pallas import tpu_sc as plsc

N = 10000
NP = 10240
E = 160000
D_FEAT = 128
D_HID = 64
N_CLS = 32

NC = 2
NS = 16
CH = 128
EPAD = 163840
NCH = EPAD // NS // CH
EPT = EPAD // NS
HALF = NP // 2
ZB = 8192
NZ = NP * NP // 32 // ZB
APAD = 16
RPS = NP // NS
RPH = HALF // NS
BM = 1024
G = NP // BM


@functools.lru_cache(maxsize=None)
def _mesh():
    return plsc.VectorSubcoreMesh(core_axis_name="c", subcore_axis_name="s",
                                  num_cores=NC, num_subcores=NS)


RPT = NP // 32
WPR = NP // 32
BITS = RPT * WPR
DUMP = BITS


@functools.lru_cache(maxsize=None)
def _sc_bits_kernel():
  @functools.partial(
    pl.kernel,
    out_type=jax.ShapeDtypeStruct((NP * WPR,), jnp.int32),
    mesh=_mesh(),
    compiler_params=pltpu.CompilerParams(needs_layout_passes=False),
    scratch_types=[
        pltpu.VMEM((NCH // 2, CH), jnp.int32),
        pltpu.VMEM((NCH // 2, CH), jnp.int32),
        pltpu.VMEM((BITS + 16,), jnp.int32),
        pltpu.VMEM((8192,), jnp.int32),
        pltpu.SemaphoreType.DMA,
    ],
  )
  def _sc_bits(src_hbm, dst_hbm, pow_hbm, words_hbm,
               src2, dst2, bitbuf, powtab, sem):
    c = lax.axis_index("c")
    s = lax.axis_index("s")
    wid = s * NC + c
    r0 = wid * RPT

    pltpu.sync_copy(pow_hbm, powtab)

    def _bz(i, carry):
        bitbuf[pl.ds(i * 16, 16)] = jnp.zeros((16,), jnp.int32)
        return carry
    lax.fori_loop(0, (BITS + 16) // 16, _bz, 0)

    for sl in range(32):
        pltpu.sync_copy(src_hbm.at[pl.ds(sl * (NCH // 2), NCH // 2)], src2)
        pltpu.sync_copy(dst_hbm.at[pl.ds(sl * (NCH // 2), NCH // 2)], dst2)

        def _scan(i, carry):
            j = i // 8
            jj = i - j * 8
            sv = src2[j, pl.ds(jj * 16, 16)]
            dv = dst2[j, pl.ds(jj * 16, 16)]
            inr = (dv >= r0) & (dv < r0 + RPT)

            cnt = plsc.all_reduce_population_count(inr)

            @pl.when(cnt[0] > 0)
            def _():
                wv = jnp.where(inr, (dv - r0) * WPR + (sv >> 5), DUMP)
                rv = (sv & 31) * 16 + (wv & 15)
                for q in range(16):
                    w = wv[q]
                    base = (w >> 4) * 16
                    add = powtab[pl.ds(rv[q] * 16, 16)]
                    bitbuf[pl.ds(base, 16)] = bitbuf[pl.ds(base, 16)] | add
            return carry
        lax.fori_loop(0, (NCH // 2) * 8, _scan, 0)

    pltpu.sync_copy(bitbuf.at[pl.ds(0, BITS)],
                    words_hbm.at[pl.ds(r0 * WPR, BITS)])

  return _sc_bits


@functools.lru_cache(maxsize=None)
def _sc_deg_kernel():
  @functools.partial(
    pl.kernel,
    out_type=(
        jax.ShapeDtypeStruct((NP,), jnp.float32),
        jax.ShapeDtypeStruct((NS * (NP + APAD),), jnp.float32),
    ),
    mesh=_mesh(),
    scratch_types=[
        pltpu.VMEM((NCH, CH), jnp.int32),
        pltpu.VMEM((NP + 2 * APAD,), jnp.float32),
        pltpu.VMEM((16, 16), jnp.float32),
        pltpu.VMEM((NS * RPS,), jnp.float32),
        pltpu.VMEM((RPS,), jnp.float32),
        pltpu.SemaphoreType.DMA,
    ],
  )
  def _sc_deg(dst_hbm, eye_hbm, deg_hbm, sh,
              dst2, hist, eyeb, redbuf, res, sem):
    c = lax.axis_index("c")
    s = lax.axis_index("s")

    @pl.when(c == 0)
    def _hist():
        pltpu.sync_copy(dst_hbm.at[pl.ds(s * NCH, NCH)], dst2)
        pltpu.sync_copy(eye_hbm, eyeb)

        def _hz(i, carry):
            hist[pl.ds(i * 16, 16)] = jnp.zeros((16,), jnp.float32)
            return carry
        lax.fori_loop(0, (NP + 2 * APAD) // 16, _hz, 0)

        def _acc(i, carry):
            j = i // 8
            jj = i - j * 8
            d16 = dst2[j, pl.ds(jj * 16, 16)]
            for q in range(16):
                d = d16[q]
                inc = eyeb[d & 15]
                base = (d >> 4) * 16
                hist[pl.ds(base, 16)] = hist[pl.ds(base, 16)] + inc
            return carry
        lax.fori_loop(0, NCH * 8, _acc, 0)
        pltpu.sync_copy(hist.at[pl.ds(0, NP + APAD)],
                        sh.at[pl.ds(s * (NP + APAD), NP + APAD)])

    plsc.subcore_barrier()

    @pl.when(c == 0)
    def _red():
        cps = [pltpu.async_copy(
                   sh.at[pl.ds(t * (NP + APAD) + s * RPS, RPS)],
                   redbuf.at[pl.ds(t * RPS, RPS)], sem)
               for t in range(NS)]
        for cp in cps:
            cp.wait()

        def _sum(i, carry):
            v = jnp.zeros((16,), jnp.float32)
            for t in range(NS):
                v = v + redbuf[pl.ds(t * RPS + i * 16, 16)]
            res[pl.ds(i * 16, 16)] = v
            return carry
        lax.fori_loop(0, RPS // 16, _sum, 0)
        pltpu.sync_copy(res, deg_hbm.at[pl.ds(s * RPS, RPS)])

  return _sc_deg


DC = 128


@functools.lru_cache(maxsize=None)
def _sc_conv_kernel():
    @functools.partial(
        pl.kernel,
        out_type=jax.ShapeDtypeStruct((NP, DC), jnp.float32),
        mesh=_mesh(),
        scratch_types=[
            pltpu.VMEM((NCH, CH), jnp.int32),
            pltpu.VMEM((NCH, CH), jnp.int32),
            pltpu.VMEM((2, CH, DC), jnp.float32),
            pltpu.VMEM((CH, DC), jnp.float32),
            pltpu.VMEM_SHARED((HALF + APAD, DC), jnp.float32),
            pltpu.SemaphoreType.DMA,
            pltpu.SemaphoreType.DMA,
        ],
    )
    def _conv(hs_hbm, src_hbm, dst_hbm, out_hbm,
              src2, dst2, rows, zrow, acc, gsem, ssem):
        c = lax.axis_index("c")
        s = lax.axis_index("s")

        def _zr(i, carry):
            for q in range(DC // 16):
                zrow[i, pl.ds(q * 16, 16)] = jnp.zeros((16,), jnp.float32)
            return carry
        lax.fori_loop(0, CH, _zr, 0)
        zb = s * ((HALF + APAD) // NS)
        pltpu.sync_copy(zrow, acc.at[pl.ds(zb, CH)])
        pltpu.sync_copy(zrow, acc.at[pl.ds(zb + CH, CH)])
        pltpu.sync_copy(zrow.at[pl.ds(0, 65)], acc.at[pl.ds(zb + 2 * CH, 65)])

        pltpu.sync_copy(src_hbm.at[pl.ds(s * NCH, NCH)], src2)
        pltpu.sync_copy(dst_hbm.at[pl.ds(s * NCH, NCH)], dst2)
        lo = c * HALF

        def _fcomp(i, carry):
            j = i // 8
            jj = i - j * 8
            dv = dst2[j, pl.ds(jj * 16, 16)]
            inhalf = (dv >= lo) & (dv < lo + HALF)
            dst2[j, pl.ds(jj * 16, 16)] = jnp.where(inhalf, dv - lo, HALF)
            return carry
        lax.fori_loop(0, NCH * 8, _fcomp, 0)
        plsc.subcore_barrier()

        for b0 in range(0, NCH, 2):
            cps = [pltpu.async_copy(hs_hbm.at[src2.at[b0 + i]], rows.at[i],
                                    gsem)
                   for i in range(2)]
            for cp in cps:
                cp.wait()
            cps = [pltpu.async_copy(rows.at[i], acc.at[dst2.at[b0 + i]], ssem,
                                    add=True)
                   for i in range(2)]
            for cp in cps:
                cp.wait()

        plsc.subcore_barrier()
        pltpu.sync_copy(acc.at[pl.ds(s * RPH, RPH)],
                        out_hbm.at[pl.ds(c * HALF + s * RPH, RPH)])

    return _conv


UBM = 256
UWB = 128
UCB = UWB * 32
NPW = 3 * UCB


def _unpack_body(w_ref, o8_ref):
    i = pl.program_id(0)
    j = pl.program_id(1)
    w = w_ref[...]
    w3 = jnp.broadcast_to(w[:, :, None], (UBM, UWB, 32))
    sh = lax.broadcasted_iota(jnp.int32, (UBM, UWB, 32), 2)
    bits = (lax.shift_right_logical(w3, sh) & 1).reshape(UBM, UCB)
    rows = lax.broadcasted_iota(jnp.int32, (UBM, UCB), 0) + i * UBM
    cols = lax.broadcasted_iota(jnp.int32, (UBM, UCB), 1) + j * UCB
    one = (bits != 0) | (rows == cols)
    o8_ref[...] = one.astype(jnp.bfloat16)


def _tc_unpack(words):
    return pl.pallas_call(
        _unpack_body,
        grid=(NP // UBM, 3),
        in_specs=[pl.BlockSpec((UBM, UWB), lambda i, j: (i, j))],
        out_specs=pl.BlockSpec((UBM, UCB), lambda i, j: (i, j)),
        out_shape=jax.ShapeDtypeStruct((NP, NPW), jnp.bfloat16),
        compiler_params=pltpu.CompilerParams(
            dimension_semantics=("parallel", "parallel")),
    )(words)


def _twohop_body(l_ref, r_ref, sub_ref, o_ref, acc_ref):
    k = pl.program_id(2)

    @pl.when(k == 0)
    def _():
        acc_ref[...] = jnp.zeros_like(acc_ref)

    acc_ref[...] += jnp.dot(l_ref[...], r_ref[...],
                            preferred_element_type=jnp.float32)

    @pl.when(k == pl.num_programs(2) - 1)
    def _():
        o_ref[...] = (acc_ref[...] > 0.0).astype(jnp.bfloat16) - sub_ref[...]


def _tc_twohop(at_s8):
    return pl.pallas_call(
        _twohop_body,
        grid=(G, G, G),
        in_specs=[
            pl.BlockSpec((BM, BM), lambda i, j, k: (i, k)),
            pl.BlockSpec((BM, BM), lambda i, j, k: (k, j)),
            pl.BlockSpec((BM, BM), lambda i, j, k: (i, j)),
        ],
        out_specs=pl.BlockSpec((BM, BM), lambda i, j, k: (i, j)),
        out_shape=jax.ShapeDtypeStruct((NP, NP), jnp.bfloat16),
        scratch_shapes=[pltpu.VMEM((BM, BM), jnp.float32)],
        compiler_params=pltpu.CompilerParams(
            dimension_semantics=("parallel", "parallel", "arbitrary")),
    )(at_s8, at_s8, at_s8)


def _deg_body(af_ref, deg1_ref, d1_ref, d2_ref, acc_ref):
    j = pl.program_id(1)

    @pl.when(j == 0)
    def _():
        acc_ref[...] = jnp.zeros_like(acc_ref)

    acc_ref[...] += jnp.sum(af_ref[...].astype(jnp.float32), axis=1,
                            keepdims=True)

    @pl.when(j == pl.num_programs(1) - 1)
    def _():
        d2_ref[...] = lax.rsqrt(acc_ref[...] + 1.0)
        d1_ref[...] = lax.rsqrt(deg1_ref[...] + 1.0)


def _tc_deg(af2t, deg1):
    return pl.pallas_call(
        _deg_body,
        grid=(G, G),
        in_specs=[
            pl.BlockSpec((BM, BM), lambda i, j: (i, j)),
            pl.BlockSpec((BM, 1), lambda i, j: (i, 0)),
        ],
        out_specs=(
            pl.BlockSpec((BM, 1), lambda i, j: (i, 0)),
            pl.BlockSpec((BM, 1), lambda i, j: (i, 0)),
        ),
        out_shape=(
            jax.ShapeDtypeStruct((NP, 1), jnp.float32),
            jax.ShapeDtypeStruct((NP, 1), jnp.float32),
        ),
        scratch_shapes=[pltpu.VMEM((BM, 1), jnp.float32)],
        compiler_params=pltpu.CompilerParams(
            dimension_semantics=("parallel", "arbitrary")),
    )(af2t, deg1)


def _mlp_body(x_ref, w_ref, d1_ref, d2_ref, h_ref, hs_ref, g_ref):
    h = jnp.dot(x_ref[...], w_ref[...], preferred_element_type=jnp.float32)
    h_ref[...] = h
    dout = h.shape[1]
    hs_ref[...] = jnp.concatenate(
        [h * d1_ref[...], jnp.zeros((h.shape[0], DC - dout), jnp.float32)],
        axis=1)
    g_ref[...] = h * d2_ref[...]


def _tc_mlp(x, w, dinv1, dinv2):
    din, dout = w.shape
    return pl.pallas_call(
        _mlp_body,
        grid=(G,),
        in_specs=[
            pl.BlockSpec((BM, din), lambda i: (i, 0)),
            pl.BlockSpec((din, dout), lambda i: (0, 0)),
            pl.BlockSpec((BM, 1), lambda i: (i, 0)),
            pl.BlockSpec((BM, 1), lambda i: (i, 0)),
        ],
        out_specs=(
            pl.BlockSpec((BM, dout), lambda i: (i, 0)),
            pl.BlockSpec((BM, DC), lambda i: (i, 0)),
            pl.BlockSpec((BM, dout), lambda i: (i, 0)),
        ),
        out_shape=(
            jax.ShapeDtypeStruct((NP, dout), jnp.float32),
            jax.ShapeDtypeStruct((NP, DC), jnp.float32),
            jax.ShapeDtypeStruct((NP, dout), jnp.float32),
        ),
        compiler_params=pltpu.CompilerParams(
            dimension_semantics=("parallel",)),
    )(x, w, dinv1, dinv2)


def _spmm_body(l_ref, g_ref, o_ref):
    j = pl.program_id(1)

    @pl.when(j == 0)
    def _():
        o_ref[...] = jnp.zeros_like(o_ref)

    o_ref[...] += jnp.dot(l_ref[...], g_ref[...].astype(jnp.bfloat16),
                          preferred_element_type=jnp.float32)


def _tc_spmm(af2t, g):
    d = g.shape[1]
    return pl.pallas_call(
        _spmm_body,
        grid=(G, G),
        in_specs=[
            pl.BlockSpec((BM, BM), lambda i, j: (i, j)),
            pl.BlockSpec((BM, d), lambda i, j: (j, 0)),
        ],
        out_specs=pl.BlockSpec((BM, d), lambda i, j: (i, 0)),
        out_shape=jax.ShapeDtypeStruct((NP, d), jnp.float32),
        compiler_params=pltpu.CompilerParams(
            dimension_semantics=("parallel", "arbitrary")),
    )(af2t, g)


def _comb_body(m_ref, t_ref, h_ref, d1_ref, d2_ref, b_ref, r_ref):
    d1 = d1_ref[...]
    d2 = d2_ref[...]
    h = h_ref[...]
    b = b_ref[...]
    out1 = d1 * m_ref[:, :h.shape[1]] + d1 * d1 * h + b
    out2 = d2 * t_ref[...] + d2 * d2 * h + b
    r_ref[...] = jnp.maximum(jnp.concatenate([out1, out2], axis=1), 0.0)


def _tc_comb(m, t, h, dinv1, dinv2, b):
    d = h.shape[1]
    return pl.pallas_call(
        _comb_body,
        grid=(G,),
        in_specs=[
            pl.BlockSpec((BM, DC), lambda i: (i, 0)),
            pl.BlockSpec((BM, d), lambda i: (i, 0)),
            pl.BlockSpec((BM, d), lambda i: (i, 0)),
            pl.BlockSpec((BM, 1), lambda i: (i, 0)),
            pl.BlockSpec((BM, 1), lambda i: (i, 0)),
            pl.BlockSpec((1, d), lambda i: (0, 0)),
        ],
        out_specs=pl.BlockSpec((BM, 2 * d), lambda i: (i, 0)),
        out_shape=jax.ShapeDtypeStruct((NP, 2 * d), jnp.float32),
        compiler_params=pltpu.CompilerParams(
            dimension_semantics=("parallel",)),
    )(m, t, h, dinv1, dinv2, b)


def _head_body(m_ref, t_ref, h_ref, d1_ref, d2_ref, b_ref,
               wl_ref, bl_ref, o_ref):
    d1 = d1_ref[...]
    d2 = d2_ref[...]
    h = h_ref[...]
    b = b_ref[...]
    h2c = d1 * m_ref[:, :h.shape[1]] + d1 * d1 * h + b
    h2_2 = d2 * t_ref[...] + d2 * d2 * h + b
    r2 = jnp.concatenate([h2c, h2_2], axis=1)
    logits = jnp.dot(r2, wl_ref[...],
                     preferred_element_type=jnp.float32) + bl_ref[...]
    m = jnp.max(logits, axis=1, keepdims=True)
    lse = jnp.log(jnp.sum(jnp.exp(logits - m), axis=1, keepdims=True)) + m
    o_ref[...] = logits - lse


def _tc_head(m, t, h, dinv1, dinv2, b, wl, bl):
    d = h.shape[1]
    return pl.pallas_call(
        _head_body,
        grid=(G,),
        in_specs=[
            pl.BlockSpec((BM, DC), lambda i: (i, 0)),
            pl.BlockSpec((BM, d), lambda i: (i, 0)),
            pl.BlockSpec((BM, d), lambda i: (i, 0)),
            pl.BlockSpec((BM, 1), lambda i: (i, 0)),
            pl.BlockSpec((BM, 1), lambda i: (i, 0)),
            pl.BlockSpec((1, d), lambda i: (0, 0)),
            pl.BlockSpec((2 * d, N_CLS), lambda i: (0, 0)),
            pl.BlockSpec((1, N_CLS), lambda i: (0, 0)),
        ],
        out_specs=pl.BlockSpec((BM, N_CLS), lambda i: (i, 0)),
        out_shape=jax.ShapeDtypeStruct((NP, N_CLS), jnp.float32),
        compiler_params=pltpu.CompilerParams(
            dimension_semantics=("parallel",)),
    )(m, t, h, dinv1, dinv2, b, wl, bl)


def kernel(x, edge_index, W1, b1, W2, b2, Wl, bl):
    x_pad = jnp.pad(x, ((0, NP - N), (0, 0)))
    src = jnp.concatenate(
        [edge_index[0], jnp.zeros((EPAD - E,), jnp.int32)]).reshape(
            EPAD // CH, CH)
    dst = jnp.concatenate(
        [edge_index[1], jnp.full((EPAD - E,), NP, jnp.int32)]).reshape(
            EPAD // CH, CH)
    bitidx = jnp.arange(512, dtype=jnp.int32)
    powtab = jnp.where(
        (jnp.arange(16, dtype=jnp.int32)[None, :] == (bitidx % 16)[:, None]),
        (jnp.int32(1) << (bitidx // 16))[:, None], 0)
    wordsflat = _sc_bits_kernel()(src, dst, powtab.reshape(-1))
    deg1, _ = _sc_deg_kernel()(dst, jnp.eye(16, dtype=jnp.float32))
    words = jnp.pad(wordsflat.reshape(NP, NP // 32), ((0, 0), (0, 64)))
    at_s8 = _tc_unpack(words)
    af2t = _tc_twohop(at_s8)
    dinv1, dinv2 = _tc_deg(af2t, deg1.reshape(NP, 1))

    h1, hs1, g1 = _tc_mlp(x_pad, W1, dinv1, dinv2)
    msg1 = _sc_conv_kernel()(hs1, src, dst)
    t1 = _tc_spmm(af2t, g1)
    r1 = _tc_comb(msg1, t1, h1, dinv1, dinv2, b1.reshape(1, -1))

    h2, hs2, g2 = _tc_mlp(r1, W2, dinv1, dinv2)
    msg2 = _sc_conv_kernel()(hs2, src, dst)
    t2 = _tc_spmm(af2t, g2)
    out = _tc_head(msg2, t2, h2, dinv1, dinv2,
                   b2.reshape(1, -1), Wl, bl.reshape(1, -1))
    return out[:N]


# speedup vs baseline: 1.0147x; 1.0147x over previous
import functools

import jax
import jax.numpy as jnp
from jax import lax
from jax.experimental import pallas as pl
from jax.experimental.pallas import tpu as pltpu
from jax.experimental.pallas import tpu_sc as plsc

N = 10000
NP = 10240
E = 160000
D_FEAT = 128
D_HID = 64
N_CLS = 32

NC = 2
NS = 16
CH = 128
EPAD = 163840
NCH = EPAD // NS // CH
EPT = EPAD // NS
HALF = NP // 2
ZB = 8192
NZ = NP * NP // 32 // ZB
APAD = 16
RPS = NP // NS
RPH = HALF // NS
BM = 1024
G = NP // BM


@functools.lru_cache(maxsize=None)
def _mesh():
    return plsc.VectorSubcoreMesh(core_axis_name="c", subcore_axis_name="s",
                                  num_cores=NC, num_subcores=NS)


RPT = NP // 32
WPR = NP // 32
BITS = RPT * WPR
DUMP = BITS


@functools.lru_cache(maxsize=None)
def _sc_bits_kernel():
  @functools.partial(
    pl.kernel,
    out_type=jax.ShapeDtypeStruct((NP * WPR,), jnp.int32),
    mesh=_mesh(),
    compiler_params=pltpu.CompilerParams(needs_layout_passes=False),
    scratch_types=[
        pltpu.VMEM((NCH // 2, CH), jnp.int32),
        pltpu.VMEM((NCH // 2, CH), jnp.int32),
        pltpu.VMEM((BITS + 16,), jnp.int32),
        pltpu.VMEM((8192,), jnp.int32),
        pltpu.SemaphoreType.DMA,
    ],
  )
  def _sc_bits(src_hbm, dst_hbm, pow_hbm, words_hbm,
               src2, dst2, bitbuf, powtab, sem):
    c = lax.axis_index("c")
    s = lax.axis_index("s")
    wid = s * NC + c
    r0 = wid * RPT

    pltpu.sync_copy(pow_hbm, powtab)

    def _bz(i, carry):
        bitbuf[pl.ds(i * 16, 16)] = jnp.zeros((16,), jnp.int32)
        return carry
    lax.fori_loop(0, (BITS + 16) // 16, _bz, 0)

    for sl in range(32):
        pltpu.sync_copy(src_hbm.at[pl.ds(sl * (NCH // 2), NCH // 2)], src2)
        pltpu.sync_copy(dst_hbm.at[pl.ds(sl * (NCH // 2), NCH // 2)], dst2)

        def _scan(i, carry):
            j = i // 8
            jj = i - j * 8
            sv = src2[j, pl.ds(jj * 16, 16)]
            dv = dst2[j, pl.ds(jj * 16, 16)]
            inr = (dv >= r0) & (dv < r0 + RPT)

            cnt = plsc.all_reduce_population_count(inr)

            @pl.when(cnt[0] > 0)
            def _():
                wv = jnp.where(inr, (dv - r0) * WPR + (sv >> 5), DUMP)
                rv = (sv & 31) * 16 + (wv & 15)
                for q in range(16):
                    w = wv[q]
                    base = (w >> 4) * 16
                    add = powtab[pl.ds(rv[q] * 16, 16)]
                    bitbuf[pl.ds(base, 16)] = bitbuf[pl.ds(base, 16)] | add
            return carry
        lax.fori_loop(0, (NCH // 2) * 8, _scan, 0)

    pltpu.sync_copy(bitbuf.at[pl.ds(0, BITS)],
                    words_hbm.at[pl.ds(r0 * WPR, BITS)])

  return _sc_bits


@functools.lru_cache(maxsize=None)
def _sc_deg_kernel():
  @functools.partial(
    pl.kernel,
    out_type=(
        jax.ShapeDtypeStruct((NP,), jnp.float32),
        jax.ShapeDtypeStruct((NS * (NP + APAD),), jnp.float32),
    ),
    mesh=_mesh(),
    scratch_types=[
        pltpu.VMEM((NCH, CH), jnp.int32),
        pltpu.VMEM((NP + 2 * APAD,), jnp.float32),
        pltpu.VMEM((16, 16), jnp.float32),
        pltpu.VMEM((NS * RPS,), jnp.float32),
        pltpu.VMEM((RPS,), jnp.float32),
        pltpu.SemaphoreType.DMA,
    ],
  )
  def _sc_deg(dst_hbm, eye_hbm, deg_hbm, sh,
              dst2, hist, eyeb, redbuf, res, sem):
    c = lax.axis_index("c")
    s = lax.axis_index("s")

    @pl.when(c == 0)
    def _hist():
        pltpu.sync_copy(dst_hbm.at[pl.ds(s * NCH, NCH)], dst2)
        pltpu.sync_copy(eye_hbm, eyeb)

        def _hz(i, carry):
            hist[pl.ds(i * 16, 16)] = jnp.zeros((16,), jnp.float32)
            return carry
        lax.fori_loop(0, (NP + 2 * APAD) // 16, _hz, 0)

        def _acc(i, carry):
            j = i // 8
            jj = i - j * 8
            d16 = dst2[j, pl.ds(jj * 16, 16)]
            for q in range(16):
                d = d16[q]
                inc = eyeb[d & 15]
                base = (d >> 4) * 16
                hist[pl.ds(base, 16)] = hist[pl.ds(base, 16)] + inc
            return carry
        lax.fori_loop(0, NCH * 8, _acc, 0)
        pltpu.sync_copy(hist.at[pl.ds(0, NP + APAD)],
                        sh.at[pl.ds(s * (NP + APAD), NP + APAD)])

    plsc.subcore_barrier()

    @pl.when(c == 0)
    def _red():
        cps = [pltpu.async_copy(
                   sh.at[pl.ds(t * (NP + APAD) + s * RPS, RPS)],
                   redbuf.at[pl.ds(t * RPS, RPS)], sem)
               for t in range(NS)]
        for cp in cps:
            cp.wait()

        def _sum(i, carry):
            v = jnp.zeros((16,), jnp.float32)
            for t in range(NS):
                v = v + redbuf[pl.ds(t * RPS + i * 16, 16)]
            res[pl.ds(i * 16, 16)] = v
            return carry
        lax.fori_loop(0, RPS // 16, _sum, 0)
        pltpu.sync_copy(res, deg_hbm.at[pl.ds(s * RPS, RPS)])

  return _sc_deg


DC = 128


@functools.lru_cache(maxsize=None)
def _sc_conv_kernel():
    @functools.partial(
        pl.kernel,
        out_type=jax.ShapeDtypeStruct((NP, DC), jnp.float32),
        mesh=_mesh(),
        scratch_types=[
            pltpu.VMEM((NCH, CH), jnp.int32),
            pltpu.VMEM((NCH, CH), jnp.int32),
            pltpu.VMEM((3, CH, DC), jnp.float32),
            pltpu.VMEM((65, DC), jnp.float32),
            pltpu.VMEM_SHARED((HALF + APAD, DC), jnp.float32),
            pltpu.SemaphoreType.DMA,
            pltpu.SemaphoreType.DMA,
        ],
    )
    def _conv(hs_hbm, src_hbm, dst_hbm, out_hbm,
              src2, dst2, rows, zrow, acc, gsem, ssem):
        c = lax.axis_index("c")
        s = lax.axis_index("s")

        def _zr(i, carry):
            for q in range(DC // 16):
                zrow[i, pl.ds(q * 16, 16)] = jnp.zeros((16,), jnp.float32)
            return carry
        lax.fori_loop(0, 65, _zr, 0)
        zb = s * ((HALF + APAD) // NS)
        for zi in range(4):
            pltpu.sync_copy(zrow.at[pl.ds(0, 64)],
                            acc.at[pl.ds(zb + zi * 64, 64)])
        pltpu.sync_copy(zrow, acc.at[pl.ds(zb + 256, 65)])

        pltpu.sync_copy(src_hbm.at[pl.ds(s * NCH, NCH)], src2)
        pltpu.sync_copy(dst_hbm.at[pl.ds(s * NCH, NCH)], dst2)
        lo = c * HALF

        def _fcomp(i, carry):
            j = i // 8
            jj = i - j * 8
            dv = dst2[j, pl.ds(jj * 16, 16)]
            inhalf = (dv >= lo) & (dv < lo + HALF)
            dst2[j, pl.ds(jj * 16, 16)] = jnp.where(inhalf, dv - lo, HALF)
            return carry
        lax.fori_loop(0, NCH * 8, _fcomp, 0)
        plsc.subcore_barrier()

        gs = [pltpu.async_copy(hs_hbm.at[src2.at[i]], rows.at[i], gsem)
              for i in range(3)]
        for b0 in range(NCH):
            gs[b0 % 3].wait()
            sc = pltpu.async_copy(rows.at[b0 % 3], acc.at[dst2.at[b0]], ssem,
                                  add=True)
            sc.wait()
            if b0 + 3 < NCH:
                gs[b0 % 3] = pltpu.async_copy(
                    hs_hbm.at[src2.at[b0 + 3]], rows.at[b0 % 3], gsem)

        plsc.subcore_barrier()
        pltpu.sync_copy(acc.at[pl.ds(s * RPH, RPH)],
                        out_hbm.at[pl.ds(c * HALF + s * RPH, RPH)])

    return _conv


UBM = 256
UWB = 128
UCB = UWB * 32
NPW = 3 * UCB


def _unpack_body(w_ref, o8_ref):
    i = pl.program_id(0)
    j = pl.program_id(1)
    w = w_ref[...]
    w3 = jnp.broadcast_to(w[:, :, None], (UBM, UWB, 32))
    sh = lax.broadcasted_iota(jnp.int32, (UBM, UWB, 32), 2)
    bits = (lax.shift_right_logical(w3, sh) & 1).reshape(UBM, UCB)
    rows = lax.broadcasted_iota(jnp.int32, (UBM, UCB), 0) + i * UBM
    cols = lax.broadcasted_iota(jnp.int32, (UBM, UCB), 1) + j * UCB
    one = (bits != 0) | (rows == cols)
    o8_ref[...] = one.astype(jnp.int8)


def _tc_unpack(words):
    return pl.pallas_call(
        _unpack_body,
        grid=(NP // UBM, 3),
        in_specs=[pl.BlockSpec((UBM, UWB), lambda i, j: (i, j))],
        out_specs=pl.BlockSpec((UBM, UCB), lambda i, j: (i, j)),
        out_shape=jax.ShapeDtypeStruct((NP, NPW), jnp.int8),
        compiler_params=pltpu.CompilerParams(
            dimension_semantics=("parallel", "parallel")),
    )(words)


def _twohop_body(l_ref, r_ref, sub_ref, o_ref, acc_ref):
    k = pl.program_id(2)

    @pl.when(k == 0)
    def _():
        acc_ref[...] = jnp.zeros_like(acc_ref)

    acc_ref[...] += jnp.dot(l_ref[...], r_ref[...],
                            preferred_element_type=jnp.int32)

    @pl.when(k == pl.num_programs(2) - 1)
    def _():
        o_ref[...] = ((acc_ref[...] > 0).astype(jnp.bfloat16)
                      - sub_ref[...].astype(jnp.bfloat16))


def _tc_twohop(at_s8):
    return pl.pallas_call(
        _twohop_body,
        grid=(G, G, G),
        in_specs=[
            pl.BlockSpec((BM, BM), lambda i, j, k: (i, k)),
            pl.BlockSpec((BM, BM), lambda i, j, k: (k, j)),
            pl.BlockSpec((BM, BM), lambda i, j, k: (i, j)),
        ],
        out_specs=pl.BlockSpec((BM, BM), lambda i, j, k: (i, j)),
        out_shape=jax.ShapeDtypeStruct((NP, NP), jnp.bfloat16),
        scratch_shapes=[pltpu.VMEM((BM, BM), jnp.int32)],
        compiler_params=pltpu.CompilerParams(
            dimension_semantics=("parallel", "parallel", "arbitrary")),
    )(at_s8, at_s8, at_s8)


def _deg_body(af_ref, deg1_ref, d1_ref, d2_ref, acc_ref):
    j = pl.program_id(1)

    @pl.when(j == 0)
    def _():
        acc_ref[...] = jnp.zeros_like(acc_ref)

    acc_ref[...] += jnp.sum(af_ref[...].astype(jnp.float32), axis=1,
                            keepdims=True)

    @pl.when(j == pl.num_programs(1) - 1)
    def _():
        d2_ref[...] = lax.rsqrt(acc_ref[...] + 1.0)
        d1_ref[...] = lax.rsqrt(deg1_ref[...] + 1.0)


def _tc_deg(af2t, deg1):
    return pl.pallas_call(
        _deg_body,
        grid=(G, G),
        in_specs=[
            pl.BlockSpec((BM, BM), lambda i, j: (i, j)),
            pl.BlockSpec((BM, 1), lambda i, j: (i, 0)),
        ],
        out_specs=(
            pl.BlockSpec((BM, 1), lambda i, j: (i, 0)),
            pl.BlockSpec((BM, 1), lambda i, j: (i, 0)),
        ),
        out_shape=(
            jax.ShapeDtypeStruct((NP, 1), jnp.float32),
            jax.ShapeDtypeStruct((NP, 1), jnp.float32),
        ),
        scratch_shapes=[pltpu.VMEM((BM, 1), jnp.float32)],
        compiler_params=pltpu.CompilerParams(
            dimension_semantics=("parallel", "arbitrary")),
    )(af2t, deg1)


def _mlp_body(x_ref, w_ref, d1_ref, d2_ref, h_ref, hs_ref, g_ref):
    h = jnp.dot(x_ref[...], w_ref[...], preferred_element_type=jnp.float32)
    h_ref[...] = h
    dout = h.shape[1]
    hs_ref[...] = jnp.concatenate(
        [h * d1_ref[...], jnp.zeros((h.shape[0], DC - dout), jnp.float32)],
        axis=1)
    g_ref[...] = h * d2_ref[...]


def _tc_mlp(x, w, dinv1, dinv2):
    din, dout = w.shape
    return pl.pallas_call(
        _mlp_body,
        grid=(G,),
        in_specs=[
            pl.BlockSpec((BM, din), lambda i: (i, 0)),
            pl.BlockSpec((din, dout), lambda i: (0, 0)),
            pl.BlockSpec((BM, 1), lambda i: (i, 0)),
            pl.BlockSpec((BM, 1), lambda i: (i, 0)),
        ],
        out_specs=(
            pl.BlockSpec((BM, dout), lambda i: (i, 0)),
            pl.BlockSpec((BM, DC), lambda i: (i, 0)),
            pl.BlockSpec((BM, dout), lambda i: (i, 0)),
        ),
        out_shape=(
            jax.ShapeDtypeStruct((NP, dout), jnp.float32),
            jax.ShapeDtypeStruct((NP, DC), jnp.float32),
            jax.ShapeDtypeStruct((NP, dout), jnp.float32),
        ),
        compiler_params=pltpu.CompilerParams(
            dimension_semantics=("parallel",)),
    )(x, w, dinv1, dinv2)


def _spmm_body(l_ref, g_ref, o_ref):
    j = pl.program_id(1)

    @pl.when(j == 0)
    def _():
        o_ref[...] = jnp.zeros_like(o_ref)

    o_ref[...] += jnp.dot(l_ref[...], g_ref[...].astype(jnp.bfloat16),
                          preferred_element_type=jnp.float32)


def _tc_spmm(af2t, g):
    d = g.shape[1]
    return pl.pallas_call(
        _spmm_body,
        grid=(G, G),
        in_specs=[
            pl.BlockSpec((BM, BM), lambda i, j: (i, j)),
            pl.BlockSpec((BM, d), lambda i, j: (j, 0)),
        ],
        out_specs=pl.BlockSpec((BM, d), lambda i, j: (i, 0)),
        out_shape=jax.ShapeDtypeStruct((NP, d), jnp.float32),
        compiler_params=pltpu.CompilerParams(
            dimension_semantics=("parallel", "arbitrary")),
    )(af2t, g)


def _comb_body(m_ref, t_ref, h_ref, d1_ref, d2_ref, b_ref, r_ref):
    d1 = d1_ref[...]
    d2 = d2_ref[...]
    h = h_ref[...]
    b = b_ref[...]
    out1 = d1 * m_ref[:, :h.shape[1]] + d1 * d1 * h + b
    out2 = d2 * t_ref[...] + d2 * d2 * h + b
    r_ref[...] = jnp.maximum(jnp.concatenate([out1, out2], axis=1), 0.0)


def _tc_comb(m, t, h, dinv1, dinv2, b):
    d = h.shape[1]
    return pl.pallas_call(
        _comb_body,
        grid=(G,),
        in_specs=[
            pl.BlockSpec((BM, DC), lambda i: (i, 0)),
            pl.BlockSpec((BM, d), lambda i: (i, 0)),
            pl.BlockSpec((BM, d), lambda i: (i, 0)),
            pl.BlockSpec((BM, 1), lambda i: (i, 0)),
            pl.BlockSpec((BM, 1), lambda i: (i, 0)),
            pl.BlockSpec((1, d), lambda i: (0, 0)),
        ],
        out_specs=pl.BlockSpec((BM, 2 * d), lambda i: (i, 0)),
        out_shape=jax.ShapeDtypeStruct((NP, 2 * d), jnp.float32),
        compiler_params=pltpu.CompilerParams(
            dimension_semantics=("parallel",)),
    )(m, t, h, dinv1, dinv2, b)


def _head_body(m_ref, t_ref, h_ref, d1_ref, d2_ref, b_ref,
               wl_ref, bl_ref, o_ref):
    d1 = d1_ref[...]
    d2 = d2_ref[...]
    h = h_ref[...]
    b = b_ref[...]
    h2c = d1 * m_ref[:, :h.shape[1]] + d1 * d1 * h + b
    h2_2 = d2 * t_ref[...] + d2 * d2 * h + b
    r2 = jnp.concatenate([h2c, h2_2], axis=1)
    logits = jnp.dot(r2, wl_ref[...],
                     preferred_element_type=jnp.float32) + bl_ref[...]
    m = jnp.max(logits, axis=1, keepdims=True)
    lse = jnp.log(jnp.sum(jnp.exp(logits - m), axis=1, keepdims=True)) + m
    o_ref[...] = logits - lse


def _tc_head(m, t, h, dinv1, dinv2, b, wl, bl):
    d = h.shape[1]
    return pl.pallas_call(
        _head_body,
        grid=(G,),
        in_specs=[
            pl.BlockSpec((BM, DC), lambda i: (i, 0)),
            pl.BlockSpec((BM, d), lambda i: (i, 0)),
            pl.BlockSpec((BM, d), lambda i: (i, 0)),
            pl.BlockSpec((BM, 1), lambda i: (i, 0)),
            pl.BlockSpec((BM, 1), lambda i: (i, 0)),
            pl.BlockSpec((1, d), lambda i: (0, 0)),
            pl.BlockSpec((2 * d, N_CLS), lambda i: (0, 0)),
            pl.BlockSpec((1, N_CLS), lambda i: (0, 0)),
        ],
        out_specs=pl.BlockSpec((BM, N_CLS), lambda i: (i, 0)),
        out_shape=jax.ShapeDtypeStruct((NP, N_CLS), jnp.float32),
        compiler_params=pltpu.CompilerParams(
            dimension_semantics=("parallel",)),
    )(m, t, h, dinv1, dinv2, b, wl, bl)


def kernel(x, edge_index, W1, b1, W2, b2, Wl, bl):
    x_pad = jnp.pad(x, ((0, NP - N), (0, 0)))
    src = jnp.concatenate(
        [edge_index[0], jnp.zeros((EPAD - E,), jnp.int32)]).reshape(
            EPAD // CH, CH)
    dst = jnp.concatenate(
        [edge_index[1], jnp.full((EPAD - E,), NP, jnp.int32)]).reshape(
            EPAD // CH, CH)
    bitidx = jnp.arange(512, dtype=jnp.int32)
    powtab = jnp.where(
        (jnp.arange(16, dtype=jnp.int32)[None, :] == (bitidx % 16)[:, None]),
        (jnp.int32(1) << (bitidx // 16))[:, None], 0)
    wordsflat = _sc_bits_kernel()(src, dst, powtab.reshape(-1))
    deg1, _ = _sc_deg_kernel()(dst, jnp.eye(16, dtype=jnp.float32))
    words = jnp.pad(wordsflat.reshape(NP, NP // 32), ((0, 0), (0, 64)))
    at_s8 = _tc_unpack(words)
    af2t = _tc_twohop(at_s8)
    dinv1, dinv2 = _tc_deg(af2t, deg1.reshape(NP, 1))

    h1, hs1, g1 = _tc_mlp(x_pad, W1, dinv1, dinv2)
    msg1 = _sc_conv_kernel()(hs1, src, dst)
    t1 = _tc_spmm(af2t, g1)
    r1 = _tc_comb(msg1, t1, h1, dinv1, dinv2, b1.reshape(1, -1))

    h2, hs2, g2 = _tc_mlp(r1, W2, dinv1, dinv2)
    msg2 = _sc_conv_kernel()(hs2, src, dst)
    t2 = _tc_spmm(af2t, g2)
    out = _tc_head(msg2, t2, h2, dinv1, dinv2,
                   b2.reshape(1, -1), Wl, bl.reshape(1, -1))
    return out[:N]
